# Initial kernel scaffold; baseline (speedup 1.0000x reference)
import jax
import jax.numpy as jnp
from jax.experimental import pallas as pl


def kernel(enc_params, atom_representation, latent_encoding, atom_positions, box_size, proj_weights):
    raise NotImplementedError("write your pallas kernel here")


# speedup vs baseline: 7.8307x; 7.8307x over previous
import functools

import numpy as np
import jax
import jax.numpy as jnp
from jax import lax
from jax.experimental import pallas as pl
from jax.experimental.pallas import tpu as pltpu
from jax.experimental.pallas import tpu_sc as plsc

_N = 4096
_K = 32
_CL = 64
_PENC = 4
_R = 0.1
_G = 32768
_PAD = 1058
_OFFS = (0, 1, 32, 33, 1024, 1025, 1056, 1057)
_RB = 256
_HB = 512
_BSITERS = 34
_HI = 1e30

_INTERPRET = False


def _posenc_consts():
  S = np.zeros((3, 96), np.float32)
  brow = np.zeros((1, 96), np.float32)
  crow = np.zeros((1, 96), np.float32)
  for c in range(8):
    sh = ((c >> 2) & 1, (c >> 1) & 1, c & 1)
    for d in range(3):
      for a in range(4):
        col = c * 12 + d * 4 + a
        ang = float(a + 1)
        S[d, col] = 1.0
        brow[0, col] = 0.5 * ang
        crow[0, col] = (sh[d] + 1.0) * 0.5 * ang
  return jnp.asarray(S), jnp.asarray(brow), jnp.asarray(crow)


def _hist_body(pos_ref, h_ref):
  b = pl.program_id(0)
  p = pos_ref[...]
  vx = jnp.clip(jnp.floor(p[:, 0:1] * 32.0), 0.0, 31.0)
  vy = jnp.clip(jnp.floor(p[:, 1:2] * 32.0), 0.0, 31.0)
  vz = jnp.clip(jnp.floor(p[:, 2:3] * 32.0), 0.0, 31.0)
  flat = (vx * 32.0 + vy) * 32.0 + vz
  col = (lax.broadcasted_iota(jnp.int32, (_N, _HB), 1)
         + b * _HB).astype(jnp.float32)
  e = (flat == col).astype(jnp.float32)
  h_ref[...] = jnp.sum(e, axis=0).reshape(1, 1, _HB)


def _hist(pos):
  nb = _G // _HB
  out = pl.pallas_call(
      _hist_body,
      grid=(nb,),
      in_specs=[pl.BlockSpec((_N, 3), lambda i: (0, 0))],
      out_specs=pl.BlockSpec((1, 1, _HB), lambda i: (i, 0, 0)),
      out_shape=jax.ShapeDtypeStruct((nb, 1, _HB), jnp.float32),
      interpret=_INTERPRET,
  )(pos)
  return out.reshape(_G)


def _prep_body(enc_ref, d8_ref, latcat_ref, wstack_ref, uv_ref):
  hp = jax.lax.Precision.HIGHEST
  wstack = wstack_ref[...]
  u = jnp.dot(latcat_ref[...], wstack, precision=hp)
  urows = [
      jnp.dot(enc_ref[...], wstack[64 * c:64 * (c + 1)], precision=hp)
      for c in range(8)
  ]
  ustack = jnp.concatenate(urows, axis=0)
  v = jnp.dot(d8_ref[...], ustack, precision=hp)
  uv_ref[...] = jnp.concatenate([u, v], axis=1)


def _prep(enc_row, d8, latcat, wstack):
  nb = 16
  blk = _G // nb
  return pl.pallas_call(
      _prep_body,
      grid=(nb,),
      in_specs=[
          pl.BlockSpec((1, _CL), lambda i: (0, 0)),
          pl.BlockSpec((blk, 8), lambda i: (i, 0)),
          pl.BlockSpec((blk, 8 * _CL), lambda i: (i, 0)),
          pl.BlockSpec((8 * _CL, _CL), lambda i: (0, 0)),
      ],
      out_specs=pl.BlockSpec((blk, 2 * _CL), lambda i: (i, 0)),
      out_shape=jax.ShapeDtypeStruct((_G, 2 * _CL), jnp.float32),
      interpret=_INTERPRET,
  )(enc_row, d8, latcat, wstack)


def _sc_gather(uv, sx, sy, sz):
  info = plsc.get_sparse_core_info()
  nc, ns = info.num_cores, info.num_subcores
  nw = nc * ns
  bpw = _N // nw

  mesh = plsc.VectorSubcoreMesh(core_axis_name="c", subcore_axis_name="s")

  @functools.partial(
      pl.kernel,
      mesh=mesh,
      out_type=jax.ShapeDtypeStruct((_N, 2 * _CL), jnp.float32),
      scratch_types=[
          pltpu.VMEM((bpw,), jnp.float32),
          pltpu.VMEM((bpw,), jnp.float32),
          pltpu.VMEM((bpw,), jnp.float32),
          pltpu.VMEM((bpw,), jnp.int32),
          pltpu.VMEM((bpw, 2 * _CL), jnp.float32),
          pltpu.SemaphoreType.DMA,
      ],
  )
  def k(uv_hbm, sx_hbm, sy_hbm, sz_hbm, out_hbm, xv, yv, zv, idxv, rows, sem):
    wid = lax.axis_index("s") * nc + lax.axis_index("c")
    base = wid * bpw
    pltpu.sync_copy(sx_hbm.at[pl.ds(base, bpw)], xv)
    pltpu.sync_copy(sy_hbm.at[pl.ds(base, bpw)], yv)
    pltpu.sync_copy(sz_hbm.at[pl.ds(base, bpw)], zv)
    for kk in range(bpw // 16):
      s = pl.ds(kk * 16, 16)
      ix = xv[s].astype(jnp.int32)
      iy = yv[s].astype(jnp.int32)
      iz = zv[s].astype(jnp.int32)
      idxv[s] = (ix * 1024 + iy * 32) + iz
    pltpu.async_copy(uv_hbm.at[idxv], rows, sem).wait()
    pltpu.sync_copy(rows, out_hbm.at[pl.ds(base, bpw)])

  return k(uv, sx, sy, sz)


def _asm_body(repr_ref, uvg_ref, s31_ref, s_ref, wpe_ref, brow_ref, crow_ref,
              a_ref):
  hp = jax.lax.Precision.HIGHEST
  s31 = s31_ref[...]
  base = jnp.floor(s31) - s31
  b96 = jnp.dot(base, s_ref[...], precision=hp)
  m = b96 * brow_ref[...] + crow_ref[...]
  pe = jnp.dot(jnp.cos(m), wpe_ref[...], precision=hp)
  uvg = uvg_ref[...]
  ug = uvg[:, :_CL]
  vg = uvg[:, _CL:]
  a_ref[...] = jnp.concatenate(
      [repr_ref[...], ug + pe, vg + pe, vg - ug], axis=1)


def _asm(arep, uvg, s31, S, wpe, brow, crow):
  return pl.pallas_call(
      _asm_body,
      out_shape=jax.ShapeDtypeStruct((_N, 448), jnp.float32),
      interpret=_INTERPRET,
  )(arep, uvg, s31, S, wpe, brow, crow)


def _attn_body(pb_ref, pt_ref, ablk_ref, at_ref, a_ref, o_ref):
  hp = jax.lax.Precision.HIGHEST
  pt = pt_ref[...]
  pb = pb_ref[...]
  dx = pb[:, 0:1] - pt[0:1, :]
  dy = pb[:, 1:2] - pt[1:2, :]
  dz = pb[:, 2:3] - pt[2:3, :]
  d2 = (dx * dx + dy * dy) + dz * dz
  dist = jnp.sqrt(d2 + 1e-16)
  inrad = dist < _R
  hi_c = jnp.float32(_HI)
  d2m = jnp.where(inrad, d2, hi_c)
  cnt_r = jnp.sum(inrad.astype(jnp.float32), axis=1, keepdims=True)

  kf = jnp.float32(_K)

  def bs(_, carry):
    lo, hi = carry
    mid = (lo + hi) * 0.5
    cnt = jnp.sum((d2m <= mid).astype(jnp.float32), axis=1, keepdims=True)
    ok = cnt <= kf
    return jnp.where(ok, mid, lo), jnp.where(ok, hi, mid)

  lo0 = jnp.zeros((_RB, 1), jnp.float32)
  hi0 = jnp.full((_RB, 1), jnp.float32(_R * _R))
  lo, _ = lax.fori_loop(0, _BSITERS, bs, (lo0, hi0))
  t = jnp.where(cnt_r <= kf, hi_c * 2.0, lo)
  sel = inrad & (d2m <= t)

  distp = dist + (dist < 1e-6).astype(jnp.float32) * 1e9
  invd2 = 1.0 / (distp * distp)

  g = jnp.dot(ablk_ref[...], at_ref[...], precision=hp)
  scale = jnp.float32(1.0) / jnp.sqrt(jnp.float32(448.0))
  logits = jnp.where(sel, g * invd2 * scale, -hi_c)
  mx = jnp.max(logits, axis=1, keepdims=True)
  e = jnp.exp(logits - mx)
  ssum = jnp.sum(e, axis=1, keepdims=True)
  w = jnp.where(sel, (e / ssum) * invd2, 0.0)
  o_ref[...] = jnp.dot(w, a_ref[...], precision=hp)


def _attn(pos, pos_t, a, a_t):
  nb = _N // _RB
  return pl.pallas_call(
      _attn_body,
      grid=(nb,),
      in_specs=[
          pl.BlockSpec((_RB, 3), lambda i: (i, 0)),
          pl.BlockSpec((3, _N), lambda i: (0, 0)),
          pl.BlockSpec((_RB, 448), lambda i: (i, 0)),
          pl.BlockSpec((448, _N), lambda i: (0, 0)),
          pl.BlockSpec((_N, 448), lambda i: (0, 0)),
      ],
      out_specs=pl.BlockSpec((_RB, 448), lambda i: (i, 0)),
      out_shape=jax.ShapeDtypeStruct((_N, 448), jnp.float32),
      interpret=_INTERPRET,
  )(pos, pos_t, a, a_t, a)


def kernel(enc_params, atom_representation, latent_encoding, atom_positions,
           box_size, proj_weights):
  pos = atom_positions
  hist = _hist(pos)
  hpad = jnp.pad(hist, (0, _PAD))
  d8 = jnp.stack([hpad[o:o + _G] for o in _OFFS], axis=1)
  latpad = jnp.pad(latent_encoding.reshape(_G, _CL), ((0, _PAD), (0, 0)))
  latcat = jnp.concatenate([latpad[o:o + _G] for o in _OFFS], axis=1)
  wstack = jnp.concatenate(
      [proj_weights[76 * c:76 * c + 64] for c in range(8)], axis=0)
  uv = _prep(enc_params.reshape(1, _CL), d8, latcat, wstack)

  s31 = pos * (31.0 / box_size)[None, :]
  uvg = _sc_gather(uv, s31[:, 0], s31[:, 1], s31[:, 2])

  wpe = jnp.concatenate(
      [proj_weights[76 * c + 64:76 * c + 76] for c in range(8)], axis=0)
  S, brow, crow = _posenc_consts()
  a = _asm(atom_representation, uvg, s31, S, wpe, brow, crow)

  return _attn(pos, pos.T, a, a.T)
